# baseline (speedup 1.0000x reference)
import jax
import jax.numpy as jnp
from jax.experimental import pallas as pl
from jax.experimental.pallas import tpu as pltpu

_N = 2048
_NUM_PHYS = 600000
_THRESH = 1.0
_BLK = 512
_Q = _BLK // 2
_E = _Q // 2
_HALF = _N // 2
_MID = _HALF - _BLK
_NSTRIP = _N // _BLK


def _notch_kernel(pk_ref, out_ref, rd_ref, ct_ref):
    pk = pk_ref[...]
    rd_ref[:, 0:_N] = pk
    rd_ref[:, _N:2 * _N] = pk
    ct_ref[...] = jnp.transpose(pk)

    lrow = jax.lax.broadcasted_iota(jnp.int32, (_E, _E), 0)
    lcol = jax.lax.broadcasted_iota(jnp.int32, (_E, _E), 1)
    upper = lcol > lrow
    wd = jnp.where(lcol == lrow, 0.5, 1.0).astype(jnp.float32)

    def strip(r, acc):
        base = r * _BLK
        cT = ct_ref[pl.ds(base, _BLK), :]

        def chain(cxc, bxc, ayc, byc, co, w):
            cxr = rd_ref[0:1, pl.ds(co, w)]
            bxr = rd_ref[1:2, pl.ds(co, w)]
            ayr = rd_ref[2:3, pl.ds(co, w)]
            byr = rd_ref[3:4, pl.ds(co, w)]
            m = jnp.minimum(jnp.minimum(cxc + bxr, cxr + bxc), _THRESH)
            dy = jnp.maximum(jnp.maximum(ayc - byr, ayr - byc), 0.0)
            p = jnp.maximum(m - dy, 0.0)
            return p * p

        def p2s(rs, co, w):
            return chain(cT[rs:rs + w, 0:1], cT[rs:rs + w, 1:2],
                         cT[rs:rs + w, 2:3], cT[rs:rs + w, 3:4], co, w)

        e256 = p2s(0, base + _Q, _Q)
        e256 += p2s(_Q, base + _HALF, _Q)
        e128 = jnp.zeros((_E, _E), jnp.float32)
        for rs, coL, coT in ((0, base, base + _HALF),
                             (_Q, base + _Q, base + _HALF + _Q)):
            e128 += p2s(rs, coL + _E, _E)
            e128 += p2s(rs + _E, coT, _E)
            e128 += jnp.where(upper, p2s(rs, coL, _E),
                              wd * p2s(rs, coT, _E))
            e128 += jnp.where(upper, p2s(rs + _E, coL + _E, _E),
                              wd * p2s(rs + _E, coT + _E, _E))
        mid = chain(cT[:, 0:1], cT[:, 1:2], cT[:, 2:3],
                    cT[:, 3:4], base + _BLK, _MID)
        return e256, e128, mid

    e256_tot = jnp.zeros((_Q, _Q), jnp.float32)
    e128_tot = jnp.zeros((_E, _E), jnp.float32)
    mid_tot = jnp.zeros((_BLK, _MID), jnp.float32)
    for _r in range(_NSTRIP):
        e2, e1, md = strip(_r, None)
        e256_tot += e2
        e128_tot += e1
        mid_tot += md
    total = (jnp.sum(e256_tot) + jnp.sum(e128_tot)) + jnp.sum(mid_tot)
    cnt = jnp.sum(pk[4:5, :])
    out_ref[0, 0] = jnp.where(cnt < 2.0, 0.0, total)


def kernel(pos, macro_mask, macro_size_x, macro_size_y):
    x = jax.lax.slice(pos, (0,), (_N,))
    y = jax.lax.slice(pos, (_NUM_PHYS,), (_NUM_PHYS + _N,))
    m = macro_mask
    mf = m.astype(jnp.float32)
    neg = jnp.where(m, jnp.float32(0.0), jnp.float32(-1e7))
    hx = macro_size_x.astype(jnp.float32) * 0.5 + neg
    hy = macro_size_y.astype(jnp.float32) * 0.5 + neg
    packed = jnp.concatenate([
        (_THRESH - x + hx).reshape(1, _N), (x + hx).reshape(1, _N),
        (y - hy).reshape(1, _N), (y + hy).reshape(1, _N),
        mf.reshape(1, _N), jnp.zeros((3, _N), jnp.float32)], axis=0)

    out = pl.pallas_call(
        _notch_kernel,
        in_specs=[pl.BlockSpec((8, _N), lambda: (0, 0))],
        out_shape=jax.ShapeDtypeStruct((1, 1), jnp.float32),
        out_specs=pl.BlockSpec(memory_space=pltpu.SMEM),
        scratch_shapes=[pltpu.VMEM((8, 2 * _N), jnp.float32),
                        pltpu.VMEM((_N, 8), jnp.float32)],
    )(packed)

    return out.reshape(())


# speedup vs baseline: 1.0087x; 1.0087x over previous
import jax
import jax.numpy as jnp
from jax.experimental import pallas as pl
from jax.experimental.pallas import tpu as pltpu

_N = 2048
_NUM_PHYS = 600000
_THRESH = 1.0
_BLK = 512
_Q = _BLK // 2
_HALF = _N // 2
_MID = _HALF - _BLK
_NSTRIP = _N // _BLK


def _notch_kernel(pk_ref, out_ref, rd_ref, ct_ref):
    pk = pk_ref[...]
    rd_ref[:, 0:_N] = pk
    rd_ref[:, _N:2 * _N] = pk
    ct_ref[...] = jnp.transpose(pk)

    lrow = jax.lax.broadcasted_iota(jnp.int32, (_Q, _Q), 0)
    lcol = jax.lax.broadcasted_iota(jnp.int32, (_Q, _Q), 1)
    upper = lcol > lrow
    wd = jnp.where(lcol == lrow, 0.5, 1.0).astype(jnp.float32)

    def strip(r, acc):
        base = r * _BLK
        cT = ct_ref[pl.ds(base, _BLK), :]

        def chain(cxc, bxc, ayc, byc, co, w):
            cxr = rd_ref[0:1, pl.ds(co, w)]
            bxr = rd_ref[1:2, pl.ds(co, w)]
            ayr = rd_ref[2:3, pl.ds(co, w)]
            byr = rd_ref[3:4, pl.ds(co, w)]
            m = jnp.minimum(jnp.minimum(cxc + bxr, cxr + bxc), _THRESH)
            dy = jnp.maximum(jnp.maximum(ayc - byr, ayr - byc), 0.0)
            p = jnp.maximum(m - dy, 0.0)
            return p * p

        def p2q(rs, co):
            return chain(cT[rs:rs + _Q, 0:1], cT[rs:rs + _Q, 1:2],
                         cT[rs:rs + _Q, 2:3], cT[rs:rs + _Q, 3:4], co, _Q)

        ends = p2q(0, base + _Q)
        ends += p2q(_Q, base + _HALF)
        ends += jnp.where(upper, p2q(0, base), wd * p2q(0, base + _HALF))
        ends += jnp.where(upper, p2q(_Q, base + _Q),
                          wd * p2q(_Q, base + _HALF + _Q))
        mid = chain(cT[:, 0:1], cT[:, 1:2], cT[:, 2:3],
                    cT[:, 3:4], base + _BLK, _MID)
        return ends, mid

    ends_tot = jnp.zeros((_Q, _Q), jnp.float32)
    mid_tot = jnp.zeros((_BLK, _MID), jnp.float32)
    for _r in range(_NSTRIP):
        e, md = strip(_r, None)
        ends_tot += e
        mid_tot += md
    total = jnp.sum(ends_tot) + jnp.sum(mid_tot)
    cnt = jnp.sum(pk[4:5, :])
    out_ref[0, 0] = jnp.where(cnt < 2.0, 0.0, total)


def kernel(pos, macro_mask, macro_size_x, macro_size_y):
    x = jax.lax.slice(pos, (0,), (_N,))
    y = jax.lax.slice(pos, (_NUM_PHYS,), (_NUM_PHYS + _N,))
    m = macro_mask
    mf = m.astype(jnp.float32)
    neg = jnp.where(m, jnp.float32(0.0), jnp.float32(-1e7))
    hx = macro_size_x.astype(jnp.float32) * 0.5 + neg
    hy = macro_size_y.astype(jnp.float32) * 0.5 + neg
    packed = jnp.concatenate([
        (_THRESH - x + hx).reshape(1, _N), (x + hx).reshape(1, _N),
        (y - hy).reshape(1, _N), (y + hy).reshape(1, _N),
        mf.reshape(1, _N), jnp.zeros((3, _N), jnp.float32)], axis=0)

    out = pl.pallas_call(
        _notch_kernel,
        in_specs=[pl.BlockSpec((8, _N), lambda: (0, 0))],
        out_shape=jax.ShapeDtypeStruct((1, 1), jnp.float32),
        out_specs=pl.BlockSpec(memory_space=pltpu.SMEM),
        scratch_shapes=[pltpu.VMEM((8, 2 * _N), jnp.float32),
                        pltpu.VMEM((_N, 8), jnp.float32)],
    )(packed)

    return out.reshape(())


# speedup vs baseline: 1.0191x; 1.0103x over previous
import jax
import jax.numpy as jnp
from jax.experimental import pallas as pl
from jax.experimental.pallas import tpu as pltpu

_N = 2048
_NUM_PHYS = 600000
_THRESH = 1.0
_BLK = 512
_Q = _BLK // 2
_HALF = _N // 2
_MID = _HALF - _BLK
_NSTRIP = _N // _BLK
_YBASE = (_NUM_PHYS // 128) * 128
_YOFF = _NUM_PHYS - _YBASE


def _notch_kernel(pos_ref, m_ref, sx_ref, sy_ref, out_ref,
                  xv_ref, yv_ref, pk_ref, rd_ref, ct_ref, sem1, sem2):
    cp1 = pltpu.make_async_copy(pos_ref.at[pl.ds(0, _N)], xv_ref, sem1)
    cp2 = pltpu.make_async_copy(pos_ref.at[pl.ds(_YBASE, _N + 128)],
                                yv_ref, sem2)
    cp1.start()
    cp2.start()
    cp1.wait()
    cp2.wait()

    row = lambda v: v.reshape(1, _N)
    mf = row(m_ref[...].astype(jnp.float32))
    neg = jnp.where(mf > 0.0, 0.0, -1e7).astype(jnp.float32)
    hx = row(sx_ref[...]) * 0.5 + neg
    hy = row(sy_ref[...]) * 0.5 + neg
    xr = row(xv_ref[...])
    yr = row(yv_ref[...][_YOFF:_YOFF + _N])
    pk_ref[0:1, :] = (_THRESH - xr) + hx
    pk_ref[1:2, :] = xr + hx
    pk_ref[2:3, :] = yr - hy
    pk_ref[3:4, :] = yr + hy
    pk_ref[4:8, :] = jnp.zeros((4, _N), jnp.float32)
    pk = pk_ref[...]
    rd_ref[:, 0:_N] = pk
    rd_ref[:, _N:2 * _N] = pk
    ct_ref[...] = jnp.transpose(pk)

    lrow = jax.lax.broadcasted_iota(jnp.int32, (_Q, _Q), 0)
    lcol = jax.lax.broadcasted_iota(jnp.int32, (_Q, _Q), 1)
    upper = lcol > lrow
    wd = jnp.where(lcol == lrow, 0.5, 1.0).astype(jnp.float32)

    def strip(r, acc):
        base = r * _BLK
        cT = ct_ref[pl.ds(base, _BLK), :]

        def chain(cxc, bxc, ayc, byc, co, w):
            cxr = rd_ref[0:1, pl.ds(co, w)]
            bxr = rd_ref[1:2, pl.ds(co, w)]
            ayr = rd_ref[2:3, pl.ds(co, w)]
            byr = rd_ref[3:4, pl.ds(co, w)]
            m = jnp.minimum(jnp.minimum(cxc + bxr, cxr + bxc), _THRESH)
            dy = jnp.maximum(jnp.maximum(ayc - byr, ayr - byc), 0.0)
            p = jnp.maximum(m - dy, 0.0)
            return p * p

        def p2q(rs, co):
            return chain(cT[rs:rs + _Q, 0:1], cT[rs:rs + _Q, 1:2],
                         cT[rs:rs + _Q, 2:3], cT[rs:rs + _Q, 3:4], co, _Q)

        ends = p2q(0, base + _Q)
        ends += p2q(_Q, base + _HALF)
        ends += jnp.where(upper, p2q(0, base), wd * p2q(0, base + _HALF))
        ends += jnp.where(upper, p2q(_Q, base + _Q),
                          wd * p2q(_Q, base + _HALF + _Q))
        mid = chain(cT[:, 0:1], cT[:, 1:2], cT[:, 2:3],
                    cT[:, 3:4], base + _BLK, _MID)
        return ends, mid

    ends_tot = jnp.zeros((_Q, _Q), jnp.float32)
    mid_tot = jnp.zeros((_BLK, _MID), jnp.float32)
    for _r in range(_NSTRIP):
        e, md = strip(_r, None)
        ends_tot += e
        mid_tot += md
    total = jnp.sum(ends_tot) + jnp.sum(mid_tot)
    cnt = jnp.sum(mf)
    out_ref[0, 0] = jnp.where(cnt < 2.0, 0.0, total)


def kernel(pos, macro_mask, macro_size_x, macro_size_y):
    out = pl.pallas_call(
        _notch_kernel,
        in_specs=[
            pl.BlockSpec(memory_space=pl.ANY),
            pl.BlockSpec((_N,), lambda: (0,)),
            pl.BlockSpec((_N,), lambda: (0,)),
            pl.BlockSpec((_N,), lambda: (0,)),
        ],
        out_shape=jax.ShapeDtypeStruct((1, 1), jnp.float32),
        out_specs=pl.BlockSpec(memory_space=pltpu.SMEM),
        scratch_shapes=[
            pltpu.VMEM((_N,), jnp.float32),
            pltpu.VMEM((_N + 128,), jnp.float32),
            pltpu.VMEM((8, _N), jnp.float32),
            pltpu.VMEM((8, 2 * _N), jnp.float32),
            pltpu.VMEM((_N, 8), jnp.float32),
            pltpu.SemaphoreType.DMA,
            pltpu.SemaphoreType.DMA,
        ],
    )(pos, macro_mask,
      macro_size_x.astype(jnp.float32), macro_size_y.astype(jnp.float32))

    return out.reshape(())


# speedup vs baseline: 1.0218x; 1.0027x over previous
import jax
import jax.numpy as jnp
from jax.experimental import pallas as pl
from jax.experimental.pallas import tpu as pltpu

_N = 2048
_NUM_PHYS = 600000
_THRESH = 1.0
_BLK = 512
_Q = _BLK // 2
_HALF = _N // 2
_MID = _HALF - _BLK
_NSTRIP = _N // _BLK
_YBASE = (_NUM_PHYS // 128) * 128
_YOFF = _NUM_PHYS - _YBASE


def _notch_kernel(pos_ref, m_ref, sx_ref, sy_ref, out_ref,
                  xv_ref, yv_ref, pk_ref, rd_ref, ct_ref, sem1, sem2):
    cp1 = pltpu.make_async_copy(pos_ref.at[pl.ds(0, _N)], xv_ref, sem1)
    cp2 = pltpu.make_async_copy(pos_ref.at[pl.ds(_YBASE, _N + 128)],
                                yv_ref, sem2)
    cp1.start()
    cp2.start()

    lrow = jax.lax.broadcasted_iota(jnp.int32, (_Q, _Q), 0)
    lcol = jax.lax.broadcasted_iota(jnp.int32, (_Q, _Q), 1)
    upper = lcol > lrow
    wd = jnp.where(lcol == lrow, 0.5, 1.0).astype(jnp.float32)

    cp1.wait()
    cp2.wait()

    row = lambda v: v.reshape(1, _N)
    mf = row(m_ref[...].astype(jnp.float32))
    neg = jnp.where(mf > 0.0, 0.0, -1e7).astype(jnp.float32)
    hx = row(sx_ref[...]) * 0.5 + neg
    hy = row(sy_ref[...]) * 0.5 + neg
    xr = row(xv_ref[...])
    yr = row(yv_ref[...][_YOFF:_YOFF + _N])
    pk_ref[0:1, :] = (_THRESH - xr) + hx
    pk_ref[1:2, :] = xr + hx
    pk_ref[2:3, :] = yr - hy
    pk_ref[3:4, :] = yr + hy
    pk_ref[4:8, :] = jnp.zeros((4, _N), jnp.float32)
    pk = pk_ref[...]
    rd_ref[:, 0:_N] = pk
    rd_ref[:, _N:2 * _N] = pk
    ct_ref[...] = jnp.transpose(pk)

    def strip(r, acc):
        base = r * _BLK
        cT = ct_ref[pl.ds(base, _BLK), :]

        def chain(cxc, bxc, ayc, byc, co, w):
            cxr = rd_ref[0:1, pl.ds(co, w)]
            bxr = rd_ref[1:2, pl.ds(co, w)]
            ayr = rd_ref[2:3, pl.ds(co, w)]
            byr = rd_ref[3:4, pl.ds(co, w)]
            m = jnp.minimum(jnp.minimum(cxc + bxr, cxr + bxc), _THRESH)
            dy = jnp.maximum(jnp.maximum(ayc - byr, ayr - byc), 0.0)
            p = jnp.maximum(m - dy, 0.0)
            return p * p

        def p2q(rs, co):
            return chain(cT[rs:rs + _Q, 0:1], cT[rs:rs + _Q, 1:2],
                         cT[rs:rs + _Q, 2:3], cT[rs:rs + _Q, 3:4], co, _Q)

        ends = p2q(0, base + _Q)
        ends += p2q(_Q, base + _HALF)
        ends += jnp.where(upper, p2q(0, base), wd * p2q(0, base + _HALF))
        ends += jnp.where(upper, p2q(_Q, base + _Q),
                          wd * p2q(_Q, base + _HALF + _Q))
        mid = chain(cT[:, 0:1], cT[:, 1:2], cT[:, 2:3],
                    cT[:, 3:4], base + _BLK, _MID)
        return ends, mid

    ends_tot = jnp.zeros((_Q, _Q), jnp.float32)
    mid_tot = jnp.zeros((_BLK, _MID), jnp.float32)
    for _r in range(_NSTRIP):
        e, md = strip(_r, None)
        ends_tot += e
        mid_tot += md
    total = jnp.sum(ends_tot) + jnp.sum(mid_tot)
    cnt = jnp.sum(mf)
    out_ref[0, 0] = jnp.where(cnt < 2.0, 0.0, total)


def kernel(pos, macro_mask, macro_size_x, macro_size_y):
    out = pl.pallas_call(
        _notch_kernel,
        in_specs=[
            pl.BlockSpec(memory_space=pl.ANY),
            pl.BlockSpec((_N,), lambda: (0,)),
            pl.BlockSpec((_N,), lambda: (0,)),
            pl.BlockSpec((_N,), lambda: (0,)),
        ],
        out_shape=jax.ShapeDtypeStruct((1, 1), jnp.float32),
        out_specs=pl.BlockSpec(memory_space=pltpu.SMEM),
        scratch_shapes=[
            pltpu.VMEM((_N,), jnp.float32),
            pltpu.VMEM((_N + 128,), jnp.float32),
            pltpu.VMEM((8, _N), jnp.float32),
            pltpu.VMEM((8, 2 * _N), jnp.float32),
            pltpu.VMEM((_N, 8), jnp.float32),
            pltpu.SemaphoreType.DMA,
            pltpu.SemaphoreType.DMA,
        ],
    )(pos, macro_mask,
      macro_size_x.astype(jnp.float32), macro_size_y.astype(jnp.float32))

    return out.reshape(())
